# baseline (speedup 1.0000x reference)
import functools
import math

import jax
import jax.numpy as jnp
import numpy as np
from jax import lax
from jax.experimental import pallas as pl
from jax.experimental.pallas import tpu as pltpu
from jax.experimental.pallas import tpu_sc as plsc

D_MODEL = 64
SEQ = 200
SCALE = 8.0
NC = 2
NS = 16
NW = NC * NS
CH = 128
LANES = 16


def _pe_scaled_dup():
    position = np.arange(0, SEQ, dtype=np.float32)[:, None]
    div = np.exp(
        np.arange(0, D_MODEL, 2, dtype=np.float32) * -(math.log(10000.0) / D_MODEL)
    )
    pe = np.zeros((SEQ, D_MODEL), dtype=np.float32)
    pe[:, 0::2] = np.sin(position * div)
    pe[:, 1::2] = np.cos(position * div)
    pe = pe * np.float32(SCALE)
    return jnp.asarray(np.concatenate([pe, pe], axis=0))


def _make_sc_embed(n_rows):
    rows_per_w = n_rows // NW
    n_chunks = rows_per_w // CH
    mesh = plsc.VectorSubcoreMesh(core_axis_name="c", subcore_axis_name="s")

    @functools.partial(
        pl.kernel,
        out_type=jax.ShapeDtypeStruct((n_rows, D_MODEL), jnp.float32),
        mesh=mesh,
        compiler_params=pltpu.CompilerParams(use_tc_tiling_on_sc=False),
        scratch_types=[
            pltpu.VMEM((CH + LANES,), jnp.int32),
            pltpu.VMEM((CH,), jnp.int32),
            pltpu.VMEM((CH,), jnp.int32),
            pltpu.VMEM((CH, 2 * D_MODEL), jnp.float32),
            pltpu.VMEM((CH, D_MODEL), jnp.float32),
            pltpu.VMEM((CH, D_MODEL), jnp.float32),
            pltpu.VMEM((2 * SEQ, D_MODEL), jnp.float32),
            pltpu.SemaphoreType.DMA,
            pltpu.SemaphoreType.DMA,
        ],
    )
    def sc_embed(xf, tf, tok2, time_tab8, pe8, out,
                 xi_v, idx2_v, tidx_v, tok_v, time_v, out_v, pe_v, sem_t, sem_m):
        wid = lax.axis_index("s") * NC + lax.axis_index("c")
        base0 = wid * rows_per_w
        pltpu.sync_copy(pe8, pe_v)

        def chunk_body(c, carry):
            base = base0 + c * CH
            s_off = lax.rem(base, SEQ)
            pltpu.sync_copy(xf.at[pl.ds(base, CH)], xi_v.at[pl.ds(0, CH)])
            pltpu.sync_copy(tf.at[pl.ds(base, CH)], tidx_v)
            for k in range(CH // LANES):
                sl = pl.ds(k * LANES, LANES)
                idx2_v[sl] = lax.shift_right_logical(xi_v[sl], 1)
            ct = pltpu.async_copy(tok2.at[idx2_v], tok_v, sem_t)
            cm = pltpu.async_copy(time_tab8.at[tidx_v], time_v, sem_m)
            ct.wait()
            cm.wait()

            def row_body(r, rcarry):
                pr = s_off + r
                off = (xi_v[pl.ds(r, LANES)][0] & 1) * D_MODEL
                for j in range(D_MODEL // LANES):
                    sl = pl.ds(j * LANES, LANES)
                    out_v[r, sl] = (
                        tok_v[r, pl.ds(off + j * LANES, LANES)] * SCALE
                        + time_v[r, sl]
                        + pe_v[pr, sl]
                    )
                return rcarry

            lax.fori_loop(0, CH, row_body, 0)
            pltpu.sync_copy(out_v, out.at[pl.ds(base, CH)])
            return carry

        lax.fori_loop(0, n_chunks, chunk_body, 0)

    return sc_embed


_sc_embed_204800 = _make_sc_embed(1024 * SEQ)


def kernel(x, time, token_table, time_table):
    b, s = x.shape
    xf = x.reshape(-1)
    tf = time.reshape(-1)
    tok2 = token_table.reshape(-1, 2 * D_MODEL)
    tt8 = (time_table * jnp.float32(SCALE)).astype(jnp.float32)
    pe8 = _pe_scaled_dup()
    out = _sc_embed_204800(xf, tf, tok2, tt8, pe8)
    return out.reshape(b, s, D_MODEL)


# speedup vs baseline: 1.0155x; 1.0155x over previous
import functools
import math

import jax
import jax.numpy as jnp
import numpy as np
from jax import lax
from jax.experimental import pallas as pl
from jax.experimental.pallas import tpu as pltpu
from jax.experimental.pallas import tpu_sc as plsc

D_MODEL = 64
SEQ = 200
NT = 49
SCALE = 8.0
NC = 2
NS = 16
NW = NC * NS
CH = 128
LANES = 16


def _pe_scaled():
    position = np.arange(0, SEQ, dtype=np.float32)[:, None]
    div = np.exp(
        np.arange(0, D_MODEL, 2, dtype=np.float32) * -(math.log(10000.0) / D_MODEL)
    )
    pe = np.zeros((SEQ, D_MODEL), dtype=np.float32)
    pe[:, 0::2] = np.sin(position * div)
    pe[:, 1::2] = np.cos(position * div)
    return jnp.asarray(pe * np.float32(SCALE))


def _make_sc_embed(n_rows):
    rows_per_w = n_rows // NW
    n_chunks = rows_per_w // CH
    mesh = plsc.VectorSubcoreMesh(core_axis_name="c", subcore_axis_name="s")

    @functools.partial(
        pl.kernel,
        out_type=jax.ShapeDtypeStruct((n_rows // 2, 2 * D_MODEL), jnp.float32),
        mesh=mesh,
        compiler_params=pltpu.CompilerParams(use_tc_tiling_on_sc=True),
        scratch_types=[
            pltpu.VMEM((CH + LANES,), jnp.int32),
            pltpu.VMEM((CH,), jnp.int32),
            pltpu.VMEM((CH,), jnp.int32),
            pltpu.VMEM((CH,), jnp.int32),
            pltpu.VMEM((CH, 2 * D_MODEL), jnp.float32),
            pltpu.VMEM((CH, 2 * D_MODEL), jnp.float32),
            pltpu.VMEM((CH // 2, 2 * D_MODEL), jnp.float32),
            pltpu.SemaphoreType.DMA,
            pltpu.SemaphoreType.DMA,
        ],
    )
    def sc_embed(xf, tf, tok2, comb, out,
                 xi_v, idx2_v, t_v, ci_v, tok_v, comb_v, out_v, sem_t, sem_m):
        wid = lax.axis_index("s") * NC + lax.axis_index("c")
        base0 = wid * rows_per_w
        lane = lax.iota(jnp.int32, LANES)

        def chunk_body(c, carry):
            base = pl.multiple_of(base0 + c * CH, CH)
            s_off = lax.rem(base, SEQ)
            pltpu.sync_copy(xf.at[pl.ds(base, CH)], xi_v.at[pl.ds(0, CH)])
            pltpu.sync_copy(tf.at[pl.ds(base, CH)], t_v)
            for k in range(CH // LANES):
                sl = pl.ds(k * LANES, LANES)
                idx2_v[sl] = lax.shift_right_logical(xi_v[sl], 1)
                ci_v[sl] = (s_off + k * LANES + lane) * NT + t_v[sl]
            ct = pltpu.async_copy(tok2.at[idx2_v], tok_v, sem_t)
            cm = pltpu.async_copy(comb.at[ci_v], comb_v, sem_m)
            ct.wait()
            cm.wait()

            def row_body(r, rcarry):
                off = (xi_v[pl.ds(r, LANES)][0] & 1) * D_MODEL
                ohalf = (r & 1) * D_MODEL
                for j in range(D_MODEL // LANES):
                    out_v[r >> 1, pl.ds(ohalf + j * LANES, LANES)] = (
                        tok_v[r, pl.ds(off + j * LANES, LANES)] * SCALE
                        + comb_v[r, pl.ds(j * LANES, LANES)]
                    )
                return rcarry

            lax.fori_loop(0, CH, row_body, 0)
            pltpu.sync_copy(out_v, out.at[pl.ds(pl.multiple_of(base // 2, CH // 2), CH // 2)])
            return carry

        lax.fori_loop(0, n_chunks, chunk_body, 0)

    return sc_embed


_sc_embed_204800 = _make_sc_embed(1024 * SEQ)


def kernel(x, time, token_table, time_table):
    b, s = x.shape
    xf = x.reshape(-1)
    tf = time.reshape(-1)
    tok2 = token_table.reshape(-1, 2 * D_MODEL)
    pe8 = _pe_scaled()
    comb = pe8[:, None, :] + time_table[None, :, :] * jnp.float32(SCALE)
    comb = jnp.pad(comb.reshape(SEQ * NT, D_MODEL), ((0, 0), (0, D_MODEL)))
    out = _sc_embed_204800(xf, tf, tok2, comb)
    return out.reshape(b, s, D_MODEL)


# speedup vs baseline: 1.0163x; 1.0008x over previous
import functools
import math

import jax
import jax.numpy as jnp
import numpy as np
from jax import lax
from jax.experimental import pallas as pl
from jax.experimental.pallas import tpu as pltpu
from jax.experimental.pallas import tpu_sc as plsc

D_MODEL = 64
SEQ = 200
NT = 49
SCALE = 8.0
NC = 2
NS = 16
NW = NC * NS
CH = 128
LANES = 16


def _pe_scaled():
    position = np.arange(0, SEQ, dtype=np.float32)[:, None]
    div = np.exp(
        np.arange(0, D_MODEL, 2, dtype=np.float32) * -(math.log(10000.0) / D_MODEL)
    )
    pe = np.zeros((SEQ, D_MODEL), dtype=np.float32)
    pe[:, 0::2] = np.sin(position * div)
    pe[:, 1::2] = np.cos(position * div)
    return jnp.asarray(pe * np.float32(SCALE))


def _make_sc_embed(n_rows):
    rows_per_w = n_rows // NW
    n_chunks = rows_per_w // CH
    mesh = plsc.VectorSubcoreMesh(core_axis_name="c", subcore_axis_name="s")

    @functools.partial(
        pl.kernel,
        out_type=jax.ShapeDtypeStruct((n_rows // 2, 2 * D_MODEL), jnp.float32),
        mesh=mesh,
        compiler_params=pltpu.CompilerParams(use_tc_tiling_on_sc=True),
        scratch_types=[
            pltpu.VMEM((CH + LANES,), jnp.int32),
            pltpu.VMEM((CH,), jnp.int32),
            pltpu.VMEM((CH,), jnp.int32),
            pltpu.VMEM((CH,), jnp.int32),
            pltpu.VMEM((CH, 2 * D_MODEL), jnp.float32),
            pltpu.VMEM((CH, 2 * D_MODEL), jnp.float32),
            pltpu.VMEM((CH // 2, 2 * D_MODEL), jnp.float32),
            pltpu.SemaphoreType.DMA,
            pltpu.SemaphoreType.DMA,
        ],
    )
    def sc_embed(xf, tf, tok2, comb, out,
                 xi_v, idx2_v, t_v, ci_v, tok_v, comb_v, out_v, sem_t, sem_m):
        wid = lax.axis_index("s") * NC + lax.axis_index("c")
        base0 = wid * rows_per_w
        lane = lax.iota(jnp.int32, LANES)

        def chunk_body(c, carry):
            base = pl.multiple_of(base0 + c * CH, CH)
            s_off = lax.rem(base, SEQ)
            pltpu.sync_copy(xf.at[pl.ds(base, CH)], xi_v.at[pl.ds(0, CH)])
            pltpu.sync_copy(tf.at[pl.ds(base, CH)], t_v)
            for k in range(CH // LANES):
                sl = pl.ds(k * LANES, LANES)
                idx2_v[sl] = lax.shift_right_logical(xi_v[sl], 1)
                ci_v[sl] = lax.rem(s_off + k * LANES + lane, SEQ) * NT + t_v[sl]
            ct = pltpu.async_copy(tok2.at[idx2_v], tok_v, sem_t)
            cm = pltpu.async_copy(comb.at[ci_v], comb_v, sem_m)
            ct.wait()
            cm.wait()

            def row_body(r, rcarry):
                off = (xi_v[pl.ds(r, LANES)][0] & 1) * D_MODEL
                ohalf = (r & 1) * D_MODEL
                for j in range(D_MODEL // LANES):
                    out_v[r >> 1, pl.ds(ohalf + j * LANES, LANES)] = (
                        tok_v[r, pl.ds(off + j * LANES, LANES)] * SCALE
                        + comb_v[r, pl.ds(j * LANES, LANES)]
                    )
                return rcarry

            lax.fori_loop(0, CH, row_body, 0)
            pltpu.sync_copy(out_v, out.at[pl.ds(pl.multiple_of(base // 2, CH // 2), CH // 2)])
            return carry

        lax.fori_loop(0, n_chunks, chunk_body, 0)

    return sc_embed


_sc_embed_204800 = _make_sc_embed(1024 * SEQ)


def kernel(x, time, token_table, time_table):
    b, s = x.shape
    xf = x.reshape(-1)
    tf = time.reshape(-1)
    tok2 = token_table.reshape(-1, 2 * D_MODEL)
    pe8 = _pe_scaled()
    comb = pe8[:, None, :] + time_table[None, :, :] * jnp.float32(SCALE)
    comb = jnp.pad(comb.reshape(SEQ * NT, D_MODEL), ((0, 0), (0, D_MODEL)))
    out = _sc_embed_204800(xf, tf, tok2, comb)
    return out.reshape(b, s, D_MODEL)


# speedup vs baseline: 1.3835x; 1.3613x over previous
import functools
import math

import jax
import jax.numpy as jnp
import numpy as np
from jax import lax
from jax.experimental import pallas as pl
from jax.experimental.pallas import tpu as pltpu
from jax.experimental.pallas import tpu_sc as plsc

D_MODEL = 64
SEQ = 200
NT = 49
SCALE = 8.0
NC = 2
NS = 16
NW = NC * NS
CH = 128
LANES = 16


def _pe_scaled():
    position = np.arange(0, SEQ, dtype=np.float32)[:, None]
    div = np.exp(
        np.arange(0, D_MODEL, 2, dtype=np.float32) * -(math.log(10000.0) / D_MODEL)
    )
    pe = np.zeros((SEQ, D_MODEL), dtype=np.float32)
    pe[:, 0::2] = np.sin(position * div)
    pe[:, 1::2] = np.cos(position * div)
    return jnp.asarray(pe * np.float32(SCALE))


def _make_sc_embed(n_rows):
    rows_per_w = n_rows // NW
    n_chunks = rows_per_w // CH
    mesh = plsc.VectorSubcoreMesh(core_axis_name="c", subcore_axis_name="s")

    @functools.partial(
        pl.kernel,
        out_type=jax.ShapeDtypeStruct((n_rows // 2, 2 * D_MODEL), jnp.float32),
        mesh=mesh,
        compiler_params=pltpu.CompilerParams(use_tc_tiling_on_sc=True),
        scratch_types=[
            pltpu.VMEM((CH + LANES,), jnp.int32),
            pltpu.VMEM((CH,), jnp.int32),
            pltpu.VMEM((CH,), jnp.int32),
            pltpu.VMEM((CH, D_MODEL), jnp.float32),
            pltpu.VMEM((CH, 2 * D_MODEL), jnp.float32),
            pltpu.VMEM((CH // 2, 2 * D_MODEL), jnp.float32),
            pltpu.SemaphoreType.DMA,
            pltpu.SemaphoreType.DMA,
        ],
    )
    def sc_embed(xf, tf, tok_tab, comb, out,
                 xi_v, t_v, ci_v, tok_v, comb_v, out_v, sem_t, sem_m):
        wid = lax.axis_index("s") * NC + lax.axis_index("c")
        base0 = wid * rows_per_w
        lane = lax.iota(jnp.int32, LANES)

        def chunk_body(c, carry):
            base = pl.multiple_of(base0 + c * CH, CH)
            s_off = lax.rem(base, SEQ)
            pltpu.sync_copy(xf.at[pl.ds(base, CH)], xi_v.at[pl.ds(0, CH)])
            pltpu.sync_copy(tf.at[pl.ds(base, CH)], t_v)
            for k in range(CH // LANES):
                sl = pl.ds(k * LANES, LANES)
                ci_v[sl] = lax.rem(s_off + k * LANES + lane, SEQ) * NT + t_v[sl]
            cm = pltpu.async_copy(comb.at[ci_v], comb_v, sem_m)

            def fetch_body(r, rcarry):
                xr = xi_v[pl.ds(r, LANES)][0]
                pltpu.async_copy(tok_tab.at[xr], tok_v.at[r], sem_t)
                return rcarry

            lax.fori_loop(0, CH, fetch_body, 0)

            def drain_body(r, rcarry):
                pltpu.make_async_copy(tok_tab.at[0], tok_v.at[0], sem_t).wait()
                return rcarry

            lax.fori_loop(0, CH, drain_body, 0)
            cm.wait()

            def row_body(r, rcarry):
                ohalf = (r & 1) * D_MODEL
                for j in range(D_MODEL // LANES):
                    sl = pl.ds(j * LANES, LANES)
                    out_v[r >> 1, pl.ds(ohalf + j * LANES, LANES)] = (
                        tok_v[r, sl] * SCALE + comb_v[r, sl]
                    )
                return rcarry

            lax.fori_loop(0, CH, row_body, 0)
            pltpu.sync_copy(out_v, out.at[pl.ds(pl.multiple_of(base // 2, CH // 2), CH // 2)])
            return carry

        lax.fori_loop(0, n_chunks, chunk_body, 0)

    return sc_embed


_sc_embed_204800 = _make_sc_embed(1024 * SEQ)


def kernel(x, time, token_table, time_table):
    b, s = x.shape
    xf = x.reshape(-1)
    tf = time.reshape(-1)
    pe8 = _pe_scaled()
    comb = pe8[:, None, :] + time_table[None, :, :] * jnp.float32(SCALE)
    comb = jnp.pad(comb.reshape(SEQ * NT, D_MODEL), ((0, 0), (0, D_MODEL)))
    out = _sc_embed_204800(xf, tf, token_table, comb)
    return out.reshape(b, s, D_MODEL)


# speedup vs baseline: 1.5683x; 1.1336x over previous
import functools
import math

import jax
import jax.numpy as jnp
import numpy as np
from jax import lax
from jax.experimental import pallas as pl
from jax.experimental.pallas import tpu as pltpu
from jax.experimental.pallas import tpu_sc as plsc

D_MODEL = 64
SEQ = 200
NT = 49
SCALE = 8.0
NC = 2
NS = 16
NW = NC * NS
CH = 128
LANES = 16


def _pe_scaled():
    position = np.arange(0, SEQ, dtype=np.float32)[:, None]
    div = np.exp(
        np.arange(0, D_MODEL, 2, dtype=np.float32) * -(math.log(10000.0) / D_MODEL)
    )
    pe = np.zeros((SEQ, D_MODEL), dtype=np.float32)
    pe[:, 0::2] = np.sin(position * div)
    pe[:, 1::2] = np.cos(position * div)
    return jnp.asarray(pe * np.float32(SCALE))


def _make_sc_embed(n_rows):
    rows_per_w = n_rows // NW
    n_chunks = rows_per_w // CH
    n_pairs = n_chunks // 2
    mesh = plsc.VectorSubcoreMesh(core_axis_name="c", subcore_axis_name="s")

    @functools.partial(
        pl.kernel,
        out_type=jax.ShapeDtypeStruct((n_rows // 2, 2 * D_MODEL), jnp.float32),
        mesh=mesh,
        compiler_params=pltpu.CompilerParams(use_tc_tiling_on_sc=True),
        scratch_types=[
            pltpu.VMEM((2, CH), jnp.int32),
            pltpu.VMEM((2, CH), jnp.int32),
            pltpu.VMEM((2, CH), jnp.int32),
            pltpu.VMEM((2, CH, D_MODEL), jnp.float32),
            pltpu.VMEM((2, CH, 2 * D_MODEL), jnp.float32),
            pltpu.VMEM((CH // 2, 2 * D_MODEL), jnp.float32),
            pltpu.SemaphoreType.DMA,
            pltpu.SemaphoreType.DMA,
            pltpu.SemaphoreType.DMA,
            pltpu.SemaphoreType.DMA,
        ],
    )
    def sc_embed(xf, tf, tok_tab, comb, out,
                 xi_v, t_v, ci_v, tok_v, comb_v, out_v,
                 sem_t0, sem_t1, sem_m0, sem_m1):
        wid = lax.axis_index("s") * NC + lax.axis_index("c")
        base0 = wid * rows_per_w
        lane = lax.iota(jnp.int32, LANES)
        sems_t = (sem_t0, sem_t1)
        sems_m = (sem_m0, sem_m1)

        def load_and_issue(c, p):
            base = pl.multiple_of(base0 + c * CH, CH)
            s_off = lax.rem(base, SEQ)
            pltpu.sync_copy(xf.at[pl.ds(base, CH)], xi_v.at[p])
            pltpu.sync_copy(tf.at[pl.ds(base, CH)], t_v.at[p])
            for k in range(CH // LANES):
                sl = pl.ds(k * LANES, LANES)
                ci_v[p, sl] = lax.rem(s_off + k * LANES + lane, SEQ) * NT + t_v[p, sl]
            pltpu.async_copy(comb.at[ci_v.at[p]], comb_v.at[p], sems_m[p])
            for k in range(CH // LANES):
                v = xi_v[p, pl.ds(k * LANES, LANES)]
                for u in range(LANES):
                    pltpu.async_copy(
                        tok_tab.at[v[u]], tok_v.at[p, k * LANES + u], sems_t[p]
                    )

        def drain(p):
            pltpu.make_async_copy(
                tok_tab.at[pl.ds(0, CH)], tok_v.at[p], sems_t[p]
            ).wait()
            pltpu.make_async_copy(
                comb.at[pl.ds(0, CH)], comb_v.at[p], sems_m[p]
            ).wait()

        def compute_store(c, p):
            base = pl.multiple_of(base0 + c * CH, CH)

            def row_body(r, rcarry):
                ohalf = (r & 1) * D_MODEL
                for j in range(D_MODEL // LANES):
                    sl = pl.ds(j * LANES, LANES)
                    out_v[r >> 1, pl.ds(ohalf + j * LANES, LANES)] = (
                        tok_v[p, r, sl] * SCALE + comb_v[p, r, sl]
                    )
                return rcarry

            lax.fori_loop(0, CH, row_body, 0)
            pltpu.sync_copy(
                out_v, out.at[pl.ds(pl.multiple_of(base // 2, CH // 2), CH // 2)]
            )

        load_and_issue(0, 0)

        def pair_body(g, carry):
            load_and_issue(2 * g + 1, 1)
            drain(0)
            compute_store(2 * g, 0)

            @pl.when(g < n_pairs - 1)
            def _():
                load_and_issue(2 * g + 2, 0)

            drain(1)
            compute_store(2 * g + 1, 1)
            return carry

        lax.fori_loop(0, n_pairs, pair_body, 0)

    return sc_embed


_sc_embed_204800 = _make_sc_embed(1024 * SEQ)


def kernel(x, time, token_table, time_table):
    b, s = x.shape
    xf = x.reshape(-1)
    tf = time.reshape(-1)
    pe8 = _pe_scaled()
    comb = pe8[:, None, :] + time_table[None, :, :] * jnp.float32(SCALE)
    comb = jnp.pad(comb.reshape(SEQ * NT, D_MODEL), ((0, 0), (0, D_MODEL)))
    out = _sc_embed_204800(xf, tf, token_table, comb)
    return out.reshape(b, s, D_MODEL)
